# baseline (speedup 1.0000x reference)
import functools

import jax
import jax.numpy as jnp
from jax import lax
from jax.experimental import pallas as pl
from jax.experimental.pallas import tpu as pltpu
from jax.experimental.pallas import tpu_sc as plsc

N = 10000
E = 320000
D = 128
C = 10
G = 64

NC = 2
NS = 16
CH = 128
NCHUNK = 80
EDGES_PER_TILE = CH * NCHUNK
E_PAD = EDGES_PER_TILE * NC * NS
NPAD = 10240
STRIPE = NPAD // NS

BN = 2000

_sc_mesh = plsc.VectorSubcoreMesh(
    core_axis_name="c", subcore_axis_name="s", num_cores=NC, num_subcores=NS
)


@functools.partial(
    pl.kernel,
    out_type=jax.ShapeDtypeStruct((NC, NPAD, D), jnp.float32),
    mesh=_sc_mesh,
    scratch_types=[
        pltpu.VMEM((CH,), jnp.int32),
        pltpu.VMEM((CH,), jnp.int32),
        pltpu.VMEM((CH, D), jnp.float32),
        pltpu.VMEM_SHARED((NPAD, D), jnp.float32),
        pltpu.SemaphoreType.DMA,
        pltpu.SemaphoreType.DMA,
    ],
)
def _sc_aggregate(h_hbm, src_hbm, dst_hbm, zeros_hbm, out_hbm,
                  sidx, didx, rows, acc, sem_g, sem_z):
    c = lax.axis_index("c")
    s = lax.axis_index("s")
    tile = c * NS + s
    zcopy = pltpu.async_copy(zeros_hbm.at[pl.ds(s * STRIPE, STRIPE)],
                             acc.at[pl.ds(s * STRIPE, STRIPE)], sem_z)
    zcopy.wait()
    plsc.subcore_barrier()

    @pl.loop(0, NCHUNK)
    def _(j):
        pltpu.sync_copy(src_hbm.at[tile].at[j], sidx)
        pltpu.sync_copy(dst_hbm.at[tile].at[j], didx)
        pltpu.async_copy(h_hbm.at[sidx], rows, sem_g).wait()
        pltpu.sync_copy(rows, acc.at[didx], add=True)

    plsc.subcore_barrier()
    pltpu.sync_copy(acc.at[pl.ds(s * STRIPE, STRIPE)],
                    out_hbm.at[c].at[pl.ds(s * STRIPE, STRIPE)])


def _combine_body(agg_ref, h_ref, wr_ref, wt_ref, br_ref, o_ref):
    a = agg_ref[0] + agg_ref[1]
    acc = jnp.dot(a, wr_ref[...], preferred_element_type=jnp.float32,
                  precision=lax.Precision.HIGHEST)
    acc = acc + jnp.dot(h_ref[...], wt_ref[...],
                        preferred_element_type=jnp.float32,
                        precision=lax.Precision.HIGHEST)
    o_ref[...] = jnp.maximum(acc + br_ref[...], 0.0)


_combine = pl.pallas_call(
    _combine_body,
    grid=(N // BN,),
    in_specs=[
        pl.BlockSpec((NC, BN, D), lambda i: (0, i, 0)),
        pl.BlockSpec((BN, D), lambda i: (i, 0)),
        pl.BlockSpec((D, D), lambda i: (0, 0)),
        pl.BlockSpec((D, D), lambda i: (0, 0)),
        pl.BlockSpec((1, D), lambda i: (0, 0)),
    ],
    out_specs=pl.BlockSpec((BN, D), lambda i: (i, 0)),
    out_shape=jax.ShapeDtypeStruct((N, D), jnp.float32),
)


def _head_body(b_ref, h_ref, w1_ref, b1_ref, w2_ref, b2_ref, o_ref, acc):
    i = pl.program_id(0)

    @pl.when(i == 0)
    def _():
        acc[...] = jnp.zeros_like(acc)

    onehot = (b_ref[0, 0][None, :]
              == lax.broadcasted_iota(jnp.int32, (G, BN), 0)).astype(jnp.float32)
    acc[...] += jnp.dot(onehot, h_ref[...], preferred_element_type=jnp.float32,
                        precision=lax.Precision.HIGHEST)

    @pl.when(i == N // BN - 1)
    def _():
        g = jnp.maximum(
            jnp.dot(acc[...], w1_ref[...], preferred_element_type=jnp.float32,
                    precision=lax.Precision.HIGHEST) + b1_ref[...], 0.0)
        z = jnp.dot(g, w2_ref[...], preferred_element_type=jnp.float32,
                    precision=lax.Precision.HIGHEST) + b2_ref[...]
        m = jnp.max(z, axis=-1, keepdims=True)
        zm = z - m
        o_ref[...] = zm - jnp.log(jnp.sum(jnp.exp(zm), axis=-1, keepdims=True))


_head = pl.pallas_call(
    _head_body,
    grid=(N // BN,),
    in_specs=[
        pl.BlockSpec((1, 1, BN), lambda i: (i, 0, 0)),
        pl.BlockSpec((BN, D), lambda i: (i, 0)),
        pl.BlockSpec((D, D), lambda i: (0, 0)),
        pl.BlockSpec((1, D), lambda i: (0, 0)),
        pl.BlockSpec((D, C), lambda i: (0, 0)),
        pl.BlockSpec((1, C), lambda i: (0, 0)),
    ],
    out_specs=pl.BlockSpec((G, C), lambda i: (0, 0)),
    out_shape=jax.ShapeDtypeStruct((G, C), jnp.float32),
    scratch_shapes=[pltpu.VMEM((G, D), jnp.float32)],
)


def kernel(x, edge_index, batch,
           Wrel0, brel0, Wroot0, Wrel1, brel1, Wroot1, Wrel2, brel2, Wroot2,
           Wrel3, brel3, Wroot3, Wrel4, brel4, Wroot4,
           fc1_W, fc1_b, fc2_W, fc2_b):
    src = edge_index[0]
    dst = edge_index[1]
    padn = E_PAD - E
    srcp = jnp.concatenate([src, jnp.zeros((padn,), jnp.int32)])
    srcp = srcp.reshape(NC * NS, NCHUNK, CH)
    dstp = jnp.concatenate([dst, jnp.full((padn,), N, jnp.int32)])
    dstp = dstp.reshape(NC * NS, NCHUNK, CH)
    zeros = jnp.zeros((NPAD, D), jnp.float32)

    h = x
    params = [(Wrel0, brel0, Wroot0), (Wrel1, brel1, Wroot1),
              (Wrel2, brel2, Wroot2), (Wrel3, brel3, Wroot3),
              (Wrel4, brel4, Wroot4)]
    for Wr, br, Wt in params:
        agg = _sc_aggregate(h, srcp, dstp, zeros)
        h = _combine(agg, h, Wr, Wt, br.reshape(1, D))

    return _head(batch.reshape(N // BN, 1, BN), h, fc1_W,
                 fc1_b.reshape(1, D), fc2_W, fc2_b.reshape(1, C))


# speedup vs baseline: 1.0000x; 1.0000x over previous
import functools

import jax
import jax.numpy as jnp
from jax import lax
from jax.experimental import pallas as pl
from jax.experimental.pallas import tpu as pltpu
from jax.experimental.pallas import tpu_sc as plsc

N = 10000
E = 320000
D = 128
C = 10
G = 64

NC = 2
NS = 16
CH = 128
NCHUNK = 80
EDGES_PER_TILE = CH * NCHUNK
E_PAD = EDGES_PER_TILE * NC * NS
NPAD = 10240
STRIPE = NPAD // NS

BN = 2000

_sc_mesh = plsc.VectorSubcoreMesh(
    core_axis_name="c", subcore_axis_name="s", num_cores=NC, num_subcores=NS
)


@functools.partial(
    pl.kernel,
    out_type=jax.ShapeDtypeStruct((NC, NPAD, D), jnp.float32),
    mesh=_sc_mesh,
    scratch_types=[
        pltpu.VMEM((CH,), jnp.int32),
        pltpu.VMEM((CH,), jnp.int32),
        pltpu.VMEM((CH, D), jnp.float32),
        pltpu.VMEM_SHARED((NPAD, D), jnp.float32),
        pltpu.SemaphoreType.DMA,
        pltpu.SemaphoreType.DMA,
    ],
)
def _sc_aggregate(h_hbm, src_hbm, dst_hbm, zeros_hbm, out_hbm,
                  sidx, didx, rows, acc, sem_g, sem_z):
    c = lax.axis_index("c")
    s = lax.axis_index("s")
    tile = c * NS + s
    zcopy = pltpu.async_copy(zeros_hbm.at[pl.ds(s * STRIPE, STRIPE)],
                             acc.at[pl.ds(s * STRIPE, STRIPE)], sem_z)
    zcopy.wait()
    plsc.subcore_barrier()

    base = tile * EDGES_PER_TILE

    @pl.loop(0, NCHUNK)
    def _(j):
        eb = base + j * CH
        pltpu.sync_copy(src_hbm.at[pl.ds(eb, CH)], sidx)
        pltpu.sync_copy(dst_hbm.at[pl.ds(eb, CH)], didx)
        pltpu.async_copy(h_hbm.at[sidx], rows, sem_g).wait()
        pltpu.sync_copy(rows, acc.at[didx], add=True)

    plsc.subcore_barrier()
    pltpu.sync_copy(acc.at[pl.ds(s * STRIPE, STRIPE)],
                    out_hbm.at[c].at[pl.ds(s * STRIPE, STRIPE)])


def _combine_body(agg_ref, h_ref, wr_ref, wt_ref, br_ref, o_ref):
    a = agg_ref[0] + agg_ref[1]
    acc = jnp.dot(a, wr_ref[...], preferred_element_type=jnp.float32,
                  precision=lax.Precision.HIGHEST)
    acc = acc + jnp.dot(h_ref[...], wt_ref[...],
                        preferred_element_type=jnp.float32,
                        precision=lax.Precision.HIGHEST)
    o_ref[...] = jnp.maximum(acc + br_ref[...], 0.0)


_combine = pl.pallas_call(
    _combine_body,
    grid=(N // BN,),
    in_specs=[
        pl.BlockSpec((NC, BN, D), lambda i: (0, i, 0)),
        pl.BlockSpec((BN, D), lambda i: (i, 0)),
        pl.BlockSpec((D, D), lambda i: (0, 0)),
        pl.BlockSpec((D, D), lambda i: (0, 0)),
        pl.BlockSpec((1, D), lambda i: (0, 0)),
    ],
    out_specs=pl.BlockSpec((BN, D), lambda i: (i, 0)),
    out_shape=jax.ShapeDtypeStruct((N, D), jnp.float32),
)


def _head_body(b_ref, h_ref, w1_ref, b1_ref, w2_ref, b2_ref, o_ref, acc):
    i = pl.program_id(0)

    @pl.when(i == 0)
    def _():
        acc[...] = jnp.zeros_like(acc)

    onehot = (b_ref[0, 0][None, :]
              == lax.broadcasted_iota(jnp.int32, (G, BN), 0)).astype(jnp.float32)
    acc[...] += jnp.dot(onehot, h_ref[...], preferred_element_type=jnp.float32,
                        precision=lax.Precision.HIGHEST)

    @pl.when(i == N // BN - 1)
    def _():
        g = jnp.maximum(
            jnp.dot(acc[...], w1_ref[...], preferred_element_type=jnp.float32,
                    precision=lax.Precision.HIGHEST) + b1_ref[...], 0.0)
        z = jnp.dot(g, w2_ref[...], preferred_element_type=jnp.float32,
                    precision=lax.Precision.HIGHEST) + b2_ref[...]
        m = jnp.max(z, axis=-1, keepdims=True)
        zm = z - m
        o_ref[...] = zm - jnp.log(jnp.sum(jnp.exp(zm), axis=-1, keepdims=True))


_head = pl.pallas_call(
    _head_body,
    grid=(N // BN,),
    in_specs=[
        pl.BlockSpec((1, 1, BN), lambda i: (i, 0, 0)),
        pl.BlockSpec((BN, D), lambda i: (i, 0)),
        pl.BlockSpec((D, D), lambda i: (0, 0)),
        pl.BlockSpec((1, D), lambda i: (0, 0)),
        pl.BlockSpec((D, C), lambda i: (0, 0)),
        pl.BlockSpec((1, C), lambda i: (0, 0)),
    ],
    out_specs=pl.BlockSpec((G, C), lambda i: (0, 0)),
    out_shape=jax.ShapeDtypeStruct((G, C), jnp.float32),
    scratch_shapes=[pltpu.VMEM((G, D), jnp.float32)],
)


def kernel(x, edge_index, batch,
           Wrel0, brel0, Wroot0, Wrel1, brel1, Wroot1, Wrel2, brel2, Wroot2,
           Wrel3, brel3, Wroot3, Wrel4, brel4, Wroot4,
           fc1_W, fc1_b, fc2_W, fc2_b):
    src = edge_index[0]
    dst = edge_index[1]
    padn = E_PAD - E
    srcp = jnp.concatenate([src, jnp.zeros((padn,), jnp.int32)])
    dstp = jnp.concatenate([dst, jnp.full((padn,), N, jnp.int32)])
    zeros = jnp.zeros((NPAD, D), jnp.float32)

    h = x
    params = [(Wrel0, brel0, Wroot0), (Wrel1, brel1, Wroot1),
              (Wrel2, brel2, Wroot2), (Wrel3, brel3, Wroot3),
              (Wrel4, brel4, Wroot4)]
    for Wr, br, Wt in params:
        agg = _sc_aggregate(h, srcp, dstp, zeros)
        h = _combine(agg, h, Wr, Wt, br.reshape(1, D))

    return _head(batch.reshape(N // BN, 1, BN), h, fc1_W,
                 fc1_b.reshape(1, D), fc2_W, fc2_b.reshape(1, C))


# speedup vs baseline: 1.5320x; 1.5319x over previous
import functools

import jax
import jax.numpy as jnp
from jax import lax
from jax.experimental import pallas as pl
from jax.experimental.pallas import tpu as pltpu
from jax.experimental.pallas import tpu_sc as plsc

N = 10000
E = 320000
D = 128
C = 10
G = 64

NC = 2
NS = 16
CH = 128
NCHUNK = 79
EDGES_PER_TILE = CH * NCHUNK
E_PAD = EDGES_PER_TILE * NC * NS
NPAD = 10240
STRIPE = NPAD // NS

BN = 2000

_sc_mesh = plsc.VectorSubcoreMesh(
    core_axis_name="c", subcore_axis_name="s", num_cores=NC, num_subcores=NS
)


@functools.partial(
    pl.kernel,
    out_type=jax.ShapeDtypeStruct((NC, NPAD, D), jnp.float32),
    mesh=_sc_mesh,
    scratch_types=[
        pltpu.VMEM((CH,), jnp.int32),
        pltpu.VMEM((CH,), jnp.int32),
        pltpu.VMEM((CH, D), jnp.float32),
        pltpu.VMEM_SHARED((NPAD, D), jnp.float32),
        pltpu.SemaphoreType.DMA,
    ],
)
def _sc_aggregate(h_hbm, src_hbm, dst_hbm, zeros_hbm, out_hbm,
                  sidx, didx, rows, acc, sem_g):
    c = lax.axis_index("c")
    s = lax.axis_index("s")
    tile = c * NS + s
    pltpu.sync_copy(zeros_hbm.at[pl.ds(s * STRIPE, STRIPE)],
                    acc.at[pl.ds(s * STRIPE, STRIPE)])
    plsc.subcore_barrier()

    base = tile * EDGES_PER_TILE

    @pl.loop(0, NCHUNK)
    def _(j):
        eb = base + j * CH
        pltpu.sync_copy(src_hbm.at[pl.ds(eb, CH)], sidx)
        pltpu.sync_copy(dst_hbm.at[pl.ds(eb, CH)], didx)
        pltpu.async_copy(h_hbm.at[sidx], rows, sem_g).wait()
        pltpu.sync_copy(rows, acc.at[didx], add=True)

    plsc.subcore_barrier()
    pltpu.sync_copy(acc.at[pl.ds(s * STRIPE, STRIPE)],
                    out_hbm.at[c].at[pl.ds(s * STRIPE, STRIPE)])


def _combine_body(agg_ref, h_ref, wr_ref, wt_ref, br_ref, o_ref):
    a = agg_ref[0] + agg_ref[1]
    acc = jnp.dot(a, wr_ref[...], preferred_element_type=jnp.float32,
                  precision=lax.Precision.HIGHEST)
    acc = acc + jnp.dot(h_ref[...], wt_ref[...],
                        preferred_element_type=jnp.float32,
                        precision=lax.Precision.HIGHEST)
    o_ref[...] = jnp.maximum(acc + br_ref[...], 0.0)


_combine = pl.pallas_call(
    _combine_body,
    grid=(N // BN,),
    in_specs=[
        pl.BlockSpec((NC, BN, D), lambda i: (0, i, 0)),
        pl.BlockSpec((BN, D), lambda i: (i, 0)),
        pl.BlockSpec((D, D), lambda i: (0, 0)),
        pl.BlockSpec((D, D), lambda i: (0, 0)),
        pl.BlockSpec((1, D), lambda i: (0, 0)),
    ],
    out_specs=pl.BlockSpec((BN, D), lambda i: (i, 0)),
    out_shape=jax.ShapeDtypeStruct((N, D), jnp.float32),
)


def _head_body(b_ref, h_ref, w1_ref, b1_ref, w2_ref, b2_ref, o_ref, acc):
    i = pl.program_id(0)

    @pl.when(i == 0)
    def _():
        acc[...] = jnp.zeros_like(acc)

    onehot = (b_ref[0, 0][None, :]
              == lax.broadcasted_iota(jnp.int32, (G, BN), 0)).astype(jnp.float32)
    acc[...] += jnp.dot(onehot, h_ref[...], preferred_element_type=jnp.float32,
                        precision=lax.Precision.HIGHEST)

    @pl.when(i == N // BN - 1)
    def _():
        g = jnp.maximum(
            jnp.dot(acc[...], w1_ref[...], preferred_element_type=jnp.float32,
                    precision=lax.Precision.HIGHEST) + b1_ref[...], 0.0)
        z = jnp.dot(g, w2_ref[...], preferred_element_type=jnp.float32,
                    precision=lax.Precision.HIGHEST) + b2_ref[...]
        m = jnp.max(z, axis=-1, keepdims=True)
        zm = z - m
        o_ref[...] = zm - jnp.log(jnp.sum(jnp.exp(zm), axis=-1, keepdims=True))


_head = pl.pallas_call(
    _head_body,
    grid=(N // BN,),
    in_specs=[
        pl.BlockSpec((1, 1, BN), lambda i: (i, 0, 0)),
        pl.BlockSpec((BN, D), lambda i: (i, 0)),
        pl.BlockSpec((D, D), lambda i: (0, 0)),
        pl.BlockSpec((1, D), lambda i: (0, 0)),
        pl.BlockSpec((D, C), lambda i: (0, 0)),
        pl.BlockSpec((1, C), lambda i: (0, 0)),
    ],
    out_specs=pl.BlockSpec((G, C), lambda i: (0, 0)),
    out_shape=jax.ShapeDtypeStruct((G, C), jnp.float32),
    scratch_shapes=[pltpu.VMEM((G, D), jnp.float32)],
)


def kernel(x, edge_index, batch,
           Wrel0, brel0, Wroot0, Wrel1, brel1, Wroot1, Wrel2, brel2, Wroot2,
           Wrel3, brel3, Wroot3, Wrel4, brel4, Wroot4,
           fc1_W, fc1_b, fc2_W, fc2_b):
    src = edge_index[0]
    dst = edge_index[1]
    padn = E_PAD - E
    srcp = jnp.concatenate([src, jnp.zeros((padn,), jnp.int32)])
    dstp = jnp.concatenate([dst, jnp.full((padn,), N, jnp.int32)])
    zeros = jnp.zeros((NPAD, D), jnp.float32)

    h = x
    params = [(Wrel0, brel0, Wroot0), (Wrel1, brel1, Wroot1),
              (Wrel2, brel2, Wroot2), (Wrel3, brel3, Wroot3),
              (Wrel4, brel4, Wroot4)]
    for Wr, br, Wt in params:
        agg = _sc_aggregate(h, srcp, dstp, zeros)
        h = _combine(agg, h, Wr, Wt, br.reshape(1, D))

    return _head(batch.reshape(N // BN, 1, BN), h, fc1_W,
                 fc1_b.reshape(1, D), fc2_W, fc2_b.reshape(1, C))
